# baseline (speedup 1.0000x reference)
import jax
import jax.numpy as jnp
from jax import lax
from jax.experimental import pallas as pl
from jax.experimental.pallas import tpu as pltpu
from jax.experimental.pallas import tpu_sc as plsc

N_NODES = 10000
N_EDGES = 320000
IN_CH = 128
OUT_CH = 128

NPAD = 10240
N_CORES = 2
N_SUBCORES = 16
SLICE = NPAD // N_SUBCORES
EP = N_EDGES // N_SUBCORES
LANES = 16
UNROLL = 5
UNROLL_Z = 10

ROW_BLK = 2560
N_BLKS = NPAD // ROW_BLK


def _sc_degree_body(col_hbm, deg0_hbm, colbuf, degbuf, accbuf,
                    tmpbuf, shared, dma_sem):
    s = lax.axis_index("s")
    wid = s

    off_e = pl.multiple_of(wid * EP, EP)
    col_cp = pltpu.async_copy(col_hbm.at[1, pl.ds(off_e, EP)], colbuf, dma_sem)

    def zero_body(i, _):
        for u in range(UNROLL_Z):
            sl = pl.ds((i * UNROLL_Z + u) * LANES, LANES)
            degbuf[sl] = jnp.zeros((LANES,), jnp.float32)
        return 0
    lax.fori_loop(0, NPAD // (LANES * UNROLL_Z), zero_body, 0)
    col_cp.wait()

    ones = jnp.ones((LANES,), jnp.float32)

    def scat_body(i, _):
        for u in range(UNROLL):
            idx = colbuf[pl.ds((i * UNROLL + u) * LANES, LANES)]
            plsc.addupdate_scatter(degbuf, [idx], ones)
        return 0
    lax.fori_loop(0, EP // (LANES * UNROLL), scat_body, 0)

    pltpu.sync_copy(degbuf, shared.at[s])
    plsc.subcore_barrier()

    off_n = pl.multiple_of(s * SLICE, SLICE)
    pltpu.sync_copy(shared.at[:, pl.ds(off_n, SLICE)], tmpbuf)

    def comb_body(j, _):
        sl = pl.ds(j * LANES, LANES)
        val = tmpbuf[0, sl]
        for t in range(1, N_SUBCORES):
            val = val + tmpbuf[t, sl]
        accbuf[sl] = val
        return 0
    lax.fori_loop(0, SLICE // LANES, comb_body, 0)

    pltpu.sync_copy(accbuf, deg0_hbm.at[pl.ds(off_n, SLICE)])


@jax.jit
def _sc_degree(col):
    return pl.kernel(
        _sc_degree_body,
        out_type=jax.ShapeDtypeStruct((NPAD,), jnp.float32),
        mesh=plsc.VectorSubcoreMesh(core_axis_name="c", subcore_axis_name="s",
                                    num_cores=1),
        compiler_params=pltpu.CompilerParams(
            use_tc_tiling_on_sc=False, needs_layout_passes=False,
            disable_bounds_checks=True, disable_semaphore_checks=True,
            skip_device_barrier=True),
        scratch_types=[
            pltpu.VMEM((EP,), jnp.int32),
            pltpu.VMEM((NPAD,), jnp.float32),
            pltpu.VMEM((SLICE,), jnp.float32),
            pltpu.VMEM((N_SUBCORES, SLICE), jnp.float32),
            pltpu.VMEM_SHARED((N_SUBCORES, NPAD), jnp.float32),
            pltpu.SemaphoreType.DMA,
        ],
    )(col)


def _tc_body(x_ref, w_ref, deg0_ref, out_ref):
    h = jnp.dot(x_ref[...], w_ref[...], preferred_element_type=jnp.float32)
    deg = deg0_ref[0]
    deg_t = deg.T
    for r in range(ROW_BLK // 128):
        m = deg_t[:, r:r + 1] > 0.0
        sl = pl.ds(r * 128, 128)
        out_ref[sl, :] = jnp.where(m, h[r * 128:(r + 1) * 128, :], 0.0)


@jax.jit
def _tc_matmul_mask(x, weight, deg0):
    deg_spec = pl.BlockSpec((1, ROW_BLK // 128, 128), lambda i: (i, 0, 0))
    return pl.pallas_call(
        _tc_body,
        grid=(N_BLKS,),
        in_specs=[
            pl.BlockSpec((ROW_BLK, IN_CH), lambda i: (i, 0)),
            pl.BlockSpec((IN_CH, OUT_CH), lambda i: (0, 0)),
            deg_spec,
        ],
        out_specs=pl.BlockSpec((ROW_BLK, OUT_CH), lambda i: (i, 0)),
        out_shape=jax.ShapeDtypeStruct((N_NODES, OUT_CH), jnp.float32),
    )(x, weight, deg0.reshape(N_BLKS, ROW_BLK // 128, 128))


def kernel(x, edge_index, weight, att):
    del att
    ei = edge_index.astype(jnp.int32)
    deg = _sc_degree(ei)
    return _tc_matmul_mask(x, weight, deg)


# speedup vs baseline: 1.0295x; 1.0295x over previous
import jax
import jax.numpy as jnp
from jax import lax
from jax.experimental import pallas as pl
from jax.experimental.pallas import tpu as pltpu
from jax.experimental.pallas import tpu_sc as plsc

N_NODES = 10000
N_EDGES = 320000
IN_CH = 128
OUT_CH = 128

NPAD = 10240
N_CORES = 2
N_SUBCORES = 16
SLICE = NPAD // N_SUBCORES
EP = N_EDGES // (N_CORES * N_SUBCORES)
LANES = 16
UNROLL = 5
UNROLL_Z = 10

ROW_BLK = 2560
N_BLKS = NPAD // ROW_BLK


def _sc_degree_body(col_hbm, deg0_hbm, deg1_hbm, colbuf, degbuf, accbuf,
                    tmpbuf, shared, dma_sem):
    c = lax.axis_index("c")
    s = lax.axis_index("s")
    wid = c * N_SUBCORES + s

    off_e = pl.multiple_of(wid * EP, EP)
    col_cp = pltpu.async_copy(col_hbm.at[1, pl.ds(off_e, EP)], colbuf, dma_sem)

    def zero_body(i, _):
        for u in range(UNROLL_Z):
            sl = pl.ds((i * UNROLL_Z + u) * LANES, LANES)
            degbuf[sl] = jnp.zeros((LANES,), jnp.float32)
        return 0
    lax.fori_loop(0, NPAD // (LANES * UNROLL_Z), zero_body, 0)
    col_cp.wait()

    ones = jnp.ones((LANES,), jnp.float32)

    def scat_body(i, _):
        for u in range(UNROLL):
            idx = colbuf[pl.ds((i * UNROLL + u) * LANES, LANES)]
            plsc.addupdate_scatter(degbuf, [idx], ones)
        return 0
    lax.fori_loop(0, EP // (LANES * UNROLL), scat_body, 0)

    pltpu.sync_copy(degbuf, shared.at[s])
    plsc.subcore_barrier()

    off_n = pl.multiple_of(s * SLICE, SLICE)
    pltpu.sync_copy(shared.at[:, pl.ds(off_n, SLICE)], tmpbuf)

    def comb_body(j, _):
        sl = pl.ds(j * LANES, LANES)
        val = tmpbuf[0, sl]
        for t in range(1, N_SUBCORES):
            val = val + tmpbuf[t, sl]
        accbuf[sl] = val
        return 0
    lax.fori_loop(0, SLICE // LANES, comb_body, 0)

    @pl.when(c == 0)
    def _():
        pltpu.sync_copy(accbuf, deg0_hbm.at[pl.ds(off_n, SLICE)])

    @pl.when(c == 1)
    def _():
        pltpu.sync_copy(accbuf, deg1_hbm.at[pl.ds(off_n, SLICE)])


@jax.jit
def _sc_degree(col):
    return pl.kernel(
        _sc_degree_body,
        out_type=(jax.ShapeDtypeStruct((NPAD,), jnp.float32),
                  jax.ShapeDtypeStruct((NPAD,), jnp.float32)),
        mesh=plsc.VectorSubcoreMesh(core_axis_name="c", subcore_axis_name="s"),
        compiler_params=pltpu.CompilerParams(
            use_tc_tiling_on_sc=False, needs_layout_passes=False),
        scratch_types=[
            pltpu.VMEM((EP,), jnp.int32),
            pltpu.VMEM((NPAD,), jnp.float32),
            pltpu.VMEM((SLICE,), jnp.float32),
            pltpu.VMEM((N_SUBCORES, SLICE), jnp.float32),
            pltpu.VMEM_SHARED((N_SUBCORES, NPAD), jnp.float32),
            pltpu.SemaphoreType.DMA,
        ],
    )(col)


def _tc_body(x_ref, w_ref, deg0_ref, deg1_ref, out_ref):
    h = jnp.dot(x_ref[...], w_ref[...], preferred_element_type=jnp.float32)
    deg = deg0_ref[0] + deg1_ref[0]
    deg_t = deg.T
    for r in range(ROW_BLK // 128):
        m = deg_t[:, r:r + 1] > 0.0
        sl = pl.ds(r * 128, 128)
        out_ref[sl, :] = jnp.where(m, h[r * 128:(r + 1) * 128, :], 0.0)


@jax.jit
def _tc_matmul_mask(x, weight, deg0, deg1):
    deg_spec = pl.BlockSpec((1, ROW_BLK // 128, 128), lambda i: (i, 0, 0))
    return pl.pallas_call(
        _tc_body,
        grid=(N_BLKS,),
        in_specs=[
            pl.BlockSpec((ROW_BLK, IN_CH), lambda i: (i, 0)),
            pl.BlockSpec((IN_CH, OUT_CH), lambda i: (0, 0)),
            deg_spec,
            deg_spec,
        ],
        out_specs=pl.BlockSpec((ROW_BLK, OUT_CH), lambda i: (i, 0)),
        out_shape=jax.ShapeDtypeStruct((N_NODES, OUT_CH), jnp.float32),
    )(x, weight, deg0.reshape(N_BLKS, ROW_BLK // 128, 128),
      deg1.reshape(N_BLKS, ROW_BLK // 128, 128))


def kernel(x, edge_index, weight, att):
    del att
    ei = edge_index.astype(jnp.int32)
    deg0, deg1 = _sc_degree(ei)
    return _tc_matmul_mask(x, weight, deg0, deg1)
